# baseline (speedup 1.0000x reference)
import jax
import jax.numpy as jnp
import numpy as np
from jax.experimental import pallas as pl
from jax.experimental.pallas import tpu as pltpu

_FORGET_RATE = 0.2
_NUM_GRADUAL = 5
_N_EPOCH = 10
_SCHED = np.ones(_N_EPOCH, np.float32) * _FORGET_RATE
_SCHED[:_NUM_GRADUAL] = np.linspace(0.0, _FORGET_RATE, _NUM_GRADUAL)

_B = 128
_V = 100000
_BR = 8
_NBLK = _B // _BR
_CH = 256
_NCH = 390
_T0 = 99840
_T1 = 99968


def _stats_kernel(lab_ref, x1_ref, x2_ref, ce1_ref, ce2_ref, dis_ref):
    lab = lab_ref[...]

    def body1(c, carry):
        m1, m2 = carry
        sl = pl.ds(pl.multiple_of(c * _CH, _CH), _CH)
        return (jnp.maximum(m1, x1_ref[:, sl]),
                jnp.maximum(m2, x2_ref[:, sl]))

    ninf = jnp.full((_BR, _CH), -jnp.inf, jnp.float32)
    m1e, m2e = jax.lax.fori_loop(0, _NCH, body1, (ninf, ninf))
    m1e = jnp.maximum(m1e[:, :128], m1e[:, 128:])
    m2e = jnp.maximum(m2e[:, :128], m2e[:, 128:])
    m1e = jnp.maximum(m1e, x1_ref[:, _T0:_T1])
    m2e = jnp.maximum(m2e, x2_ref[:, _T0:_T1])
    m1e = jnp.maximum(m1e[:, :32], jnp.max(m1e[:, 32:], axis=1, keepdims=True))
    m2e = jnp.maximum(m2e[:, :32], jnp.max(m2e[:, 32:], axis=1, keepdims=True))
    m1e = jnp.maximum(m1e, x1_ref[:, _T1:_V])
    m2e = jnp.maximum(m2e, x2_ref[:, _T1:_V])
    rm1 = jnp.max(m1e, axis=1, keepdims=True)
    rm2 = jnp.max(m2e, axis=1, keepdims=True)

    liota = jax.lax.broadcasted_iota(jnp.int32, (_BR, _CH), 1)

    def proc(x1, x2, col, carry):
        s1, s2, i1, i2, t1, t2 = carry
        s1 = s1 + jnp.exp(x1 - rm1)
        s2 = s2 + jnp.exp(x2 - rm2)
        i1 = jnp.minimum(i1, jnp.where(x1 == rm1, col, _V))
        i2 = jnp.minimum(i2, jnp.where(x2 == rm2, col, _V))
        lsel = col == lab
        t1 = t1 + jnp.where(lsel, x1, 0.0)
        t2 = t2 + jnp.where(lsel, x2, 0.0)
        return s1, s2, i1, i2, t1, t2

    def body2(c, carry):
        sl = pl.ds(pl.multiple_of(c * _CH, _CH), _CH)
        return proc(x1_ref[:, sl], x2_ref[:, sl], liota + c * _CH, carry)

    z = jnp.zeros((_BR, _CH), jnp.float32)
    iv = jnp.full((_BR, _CH), _V, jnp.int32)
    carry = jax.lax.fori_loop(0, _NCH, body2, (z, z, iv, iv, z, z))
    s1e, s2e, i1e, i2e, t1e, t2e = carry
    s1e = s1e[:, :128] + s1e[:, 128:]
    s2e = s2e[:, :128] + s2e[:, 128:]
    i1e = jnp.minimum(i1e[:, :128], i1e[:, 128:])
    i2e = jnp.minimum(i2e[:, :128], i2e[:, 128:])
    t1e = t1e[:, :128] + t1e[:, 128:]
    t2e = t2e[:, :128] + t2e[:, 128:]
    c128 = jax.lax.broadcasted_iota(jnp.int32, (_BR, 128), 1)
    s1e, s2e, i1e, i2e, t1e, t2e = proc(
        x1_ref[:, _T0:_T1], x2_ref[:, _T0:_T1], c128 + _T0,
        (s1e, s2e, i1e, i2e, t1e, t2e))
    s1e32 = s1e[:, :32] + s1e[:, 32:64] + s1e[:, 64:96] + s1e[:, 96:128]
    s2e32 = s2e[:, :32] + s2e[:, 32:64] + s2e[:, 64:96] + s2e[:, 96:128]
    i1e32 = jnp.minimum(jnp.minimum(i1e[:, :32], i1e[:, 32:64]),
                        jnp.minimum(i1e[:, 64:96], i1e[:, 96:128]))
    i2e32 = jnp.minimum(jnp.minimum(i2e[:, :32], i2e[:, 32:64]),
                        jnp.minimum(i2e[:, 64:96], i2e[:, 96:128]))
    t1e32 = t1e[:, :32] + t1e[:, 32:64] + t1e[:, 64:96] + t1e[:, 96:128]
    t2e32 = t2e[:, :32] + t2e[:, 32:64] + t2e[:, 64:96] + t2e[:, 96:128]
    c32 = jax.lax.broadcasted_iota(jnp.int32, (_BR, 32), 1)
    s1e32, s2e32, i1e32, i2e32, t1e32, t2e32 = proc(
        x1_ref[:, _T1:_V], x2_ref[:, _T1:_V], c32 + _T1,
        (s1e32, s2e32, i1e32, i2e32, t1e32, t2e32))
    s1 = jnp.sum(s1e32, axis=1, keepdims=True)
    s2 = jnp.sum(s2e32, axis=1, keepdims=True)
    a1 = jnp.min(i1e32, axis=1, keepdims=True)
    a2 = jnp.min(i2e32, axis=1, keepdims=True)
    t1 = jnp.sum(t1e32, axis=1, keepdims=True)
    t2 = jnp.sum(t2e32, axis=1, keepdims=True)

    ce1_ref[...] = rm1 + jnp.log(s1) - t1
    ce2_ref[...] = rm2 + jnp.log(s2) - t2
    dis_ref[...] = (a1 != a2).astype(jnp.int32)


def _select_kernel(rr_ref, uf_ref, l1c_ref, l2c_ref, dc_ref,
                   l1r_ref, l2r_ref, dr_ref, o1_ref, o2_ref):
    l1c = l1c_ref[...]
    l2c = l2c_ref[...]
    dc = dc_ref[...]
    l1r = l1r_ref[0:1, :]
    l2r = l2r_ref[0:1, :]
    dr = dr_ref[0:1, :]

    dcf = dc.astype(jnp.float32)
    drf = dr.astype(jnp.float32)
    D = jnp.sum(dcf)
    ridc = jax.lax.broadcasted_iota(jnp.int32, (_B, 1), 0)
    dropped = jnp.sum(jnp.where(ridc == 0, dcf, 0.0))
    L = D - dropped

    rid = jax.lax.broadcasted_iota(jnp.int32, (_B, _B), 0)
    cid = jax.lax.broadcasted_iota(jnp.int32, (_B, _B), 1)
    dr2 = jnp.broadcast_to(drf, (_B, _B))
    dc2 = jnp.broadcast_to(dcf, (_B, _B))
    pref_c = jnp.sum(jnp.where(cid < rid, dr2, 0.0), axis=1, keepdims=True)
    pref_r = jnp.sum(jnp.where(rid < cid, dc2, 0.0), axis=0, keepdims=True)
    sel_c = (dc != 0) & (pref_c < L)
    sel_r = (dr != 0) & (pref_r < L)
    sel_r2 = jnp.broadcast_to(sel_r, (_B, _B))

    cmp2 = (l2r < l2c) | ((l2r == l2c) & (cid < rid))
    rank2 = jnp.sum(jnp.where(cmp2 & sel_r2, 1.0, 0.0), axis=1, keepdims=True)
    cmp1 = (l1r < l1c) | ((l1r == l1c) & (cid < rid))
    rank1 = jnp.sum(jnp.where(cmp1 & sel_r2, 1.0, 0.0), axis=1, keepdims=True)

    rr = rr_ref[0, 0]
    nr = jnp.floor(rr * L)
    k = jnp.where(nr == 0.0, L, nr)
    keep2 = sel_c & (rank2 < k)
    keep1 = sel_c & (rank1 < k)
    loss1_upd = jnp.sum(jnp.where(keep2, l1c, 0.0)) / k
    loss2_upd = jnp.sum(jnp.where(keep1, l2c, 0.0)) / k

    uf = uf_ref[0, 0]
    us = jnp.where((dc != 0) | (uf != 0), 1.0, 0.0)
    fb1 = jnp.sum(us * l1c) / _B
    fb2 = jnp.sum(us * l2c) / _B

    o1_ref[0, 0] = jnp.where(L > 0, loss1_upd, fb1)
    o2_ref[0, 0] = jnp.where(L > 0, loss2_upd, fb2)


def _stats_call(lab, logits, logits2):
    return pl.pallas_call(
        _stats_kernel,
        grid=(_NBLK,),
        in_specs=[
            pl.BlockSpec((_BR, 1), lambda i: (i, 0)),
            pl.BlockSpec((_BR, _V), lambda i: (i, 0)),
            pl.BlockSpec((_BR, _V), lambda i: (i, 0)),
        ],
        out_specs=[
            pl.BlockSpec((_BR, 1), lambda i: (i, 0)),
            pl.BlockSpec((_BR, 1), lambda i: (i, 0)),
            pl.BlockSpec((_BR, 1), lambda i: (i, 0)),
        ],
        out_shape=[
            jax.ShapeDtypeStruct((_B, 1), jnp.float32),
            jax.ShapeDtypeStruct((_B, 1), jnp.float32),
            jax.ShapeDtypeStruct((_B, 1), jnp.int32),
        ],
        compiler_params=pltpu.CompilerParams(
            dimension_semantics=("arbitrary",)),
    )(lab, logits, logits2)


def _select_call(rr, uf, ce1, ce2, dis, ce1r, ce2r, disr):
    o1, o2 = pl.pallas_call(
        _select_kernel,
        in_specs=[
            pl.BlockSpec(memory_space=pltpu.SMEM),
            pl.BlockSpec(memory_space=pltpu.SMEM),
            pl.BlockSpec((_B, 1), lambda: (0, 0)),
            pl.BlockSpec((_B, 1), lambda: (0, 0)),
            pl.BlockSpec((_B, 1), lambda: (0, 0)),
            pl.BlockSpec((8, _B), lambda: (0, 0)),
            pl.BlockSpec((8, _B), lambda: (0, 0)),
            pl.BlockSpec((8, _B), lambda: (0, 0)),
        ],
        out_specs=[
            pl.BlockSpec(memory_space=pltpu.SMEM),
            pl.BlockSpec(memory_space=pltpu.SMEM),
        ],
        out_shape=[
            jax.ShapeDtypeStruct((1, 1), jnp.float32),
            jax.ShapeDtypeStruct((1, 1), jnp.float32),
        ],
    )(rr, uf, ce1, ce2, dis, ce1r, ce2r, disr)
    return o1, o2


def kernel(logits, logits2, labels, epoch, ind, step):
    lab = labels.astype(jnp.int32).reshape(_B, 1)
    ce1, ce2, dis = _stats_call(lab, logits, logits2)
    rr = (1.0 - jnp.asarray(_SCHED)[epoch]).astype(jnp.float32).reshape(1, 1)
    uf = (jnp.asarray(step) < 5000).astype(jnp.int32).reshape(1, 1)
    ce1r = jnp.broadcast_to(ce1.reshape(1, _B), (8, _B))
    ce2r = jnp.broadcast_to(ce2.reshape(1, _B), (8, _B))
    disr = jnp.broadcast_to(dis.reshape(1, _B), (8, _B))
    o1, o2 = _select_call(rr, uf, ce1, ce2, dis, ce1r, ce2r, disr)
    return o1.reshape(()), o2.reshape(())


# speedup vs baseline: 1.9183x; 1.9183x over previous
import functools

import jax
import jax.numpy as jnp
import numpy as np
from jax import lax
from jax.experimental import pallas as pl
from jax.experimental.pallas import tpu as pltpu
from jax.experimental.pallas import tpu_sc as plsc

_FORGET_RATE = 0.2
_NUM_GRADUAL = 5
_N_EPOCH = 10
_SCHED = np.ones(_N_EPOCH, np.float32) * _FORGET_RATE
_SCHED[:_NUM_GRADUAL] = np.linspace(0.0, _FORGET_RATE, _NUM_GRADUAL)

_B = 128
_V = 100000
_BV = 8192
_NBLK = -(-_V // _BV)
_LL = 16
_NROWS = _V // _LL


def _stats_kernel(x1_ref, x2_ref, lse1_ref, lse2_ref, dis_ref,
                  m1, s1, a1, m2, s2, a2):
    j = pl.program_id(0)

    @pl.when(j == 0)
    def _init():
        for m, s, a in ((m1, s1, a1), (m2, s2, a2)):
            m[...] = jnp.full((_B, 1), -jnp.inf, jnp.float32)
            s[...] = jnp.zeros((_B, 1), jnp.float32)
            a[...] = jnp.zeros((_B, 1), jnp.int32)

    col = j * _BV + jax.lax.broadcasted_iota(jnp.int32, (_B, _BV), 1)

    def upd(x_raw, m, s, a):
        x = jnp.where(col < _V, x_raw, -jnp.inf)
        bmax = jnp.max(x, axis=1, keepdims=True)
        bidx = jnp.min(jnp.where(x == bmax, col, _V), axis=1, keepdims=True)
        m_old = m[...]
        m_new = jnp.maximum(m_old, bmax)
        s[...] = s[...] * jnp.exp(m_old - m_new) + jnp.sum(
            jnp.exp(x - m_new), axis=1, keepdims=True)
        a[...] = jnp.where(bmax > m_old, bidx, a[...])
        m[...] = m_new

    upd(x1_ref[...], m1, s1, a1)
    upd(x2_ref[...], m2, s2, a2)

    @pl.when(j == _NBLK - 1)
    def _fin():
        lse1_ref[...] = m1[...] + jnp.log(s1[...])
        lse2_ref[...] = m2[...] + jnp.log(s2[...])
        dis_ref[...] = (a1[...] != a2[...]).astype(jnp.int32)


def _stats_call(logits, logits2):
    return pl.pallas_call(
        _stats_kernel,
        grid=(_NBLK,),
        in_specs=[
            pl.BlockSpec((_B, _BV), lambda j: (0, j)),
            pl.BlockSpec((_B, _BV), lambda j: (0, j)),
        ],
        out_specs=[
            pl.BlockSpec((_B, 1), lambda j: (0, 0)),
            pl.BlockSpec((_B, 1), lambda j: (0, 0)),
            pl.BlockSpec((_B, 1), lambda j: (0, 0)),
        ],
        out_shape=[
            jax.ShapeDtypeStruct((_B, 1), jnp.float32),
            jax.ShapeDtypeStruct((_B, 1), jnp.float32),
            jax.ShapeDtypeStruct((_B, 1), jnp.int32),
        ],
        scratch_shapes=[
            pltpu.VMEM((_B, 1), jnp.float32),
            pltpu.VMEM((_B, 1), jnp.float32),
            pltpu.VMEM((_B, 1), jnp.int32),
            pltpu.VMEM((_B, 1), jnp.float32),
            pltpu.VMEM((_B, 1), jnp.float32),
            pltpu.VMEM((_B, 1), jnp.int32),
        ],
        compiler_params=pltpu.CompilerParams(
            dimension_semantics=("arbitrary",)),
    )(logits, logits2)


def _gather_call(lab1d, tab1, tab2):
    mesh = plsc.VectorSubcoreMesh(core_axis_name="c", subcore_axis_name="s")

    @functools.partial(
        pl.kernel, mesh=mesh,
        compiler_params=pltpu.CompilerParams(use_tc_tiling_on_sc=False),
        out_type=[jax.ShapeDtypeStruct((_B, _LL), jnp.float32),
                  jax.ShapeDtypeStruct((_B, _LL), jnp.float32)],
        scratch_types=[
            pltpu.VMEM((_B,), jnp.int32),
            pltpu.VMEM((_B,), jnp.int32),
            pltpu.VMEM((_B, _LL), jnp.float32),
            pltpu.VMEM((_B, _LL), jnp.float32),
            pltpu.SemaphoreType.DMA,
        ],
    )
    def k(lab_hbm, t1_hbm, t2_hbm, o1_hbm, o2_hbm,
          lab_v, idx_v, r1_v, r2_v, sem):
        cid = lax.axis_index("c")
        sid = lax.axis_index("s")

        @pl.when(jnp.logical_and(cid == 0, sid == 0))
        def _():
            pltpu.sync_copy(lab_hbm, lab_v)
            for c in range(_B // 16):
                sl = pl.ds(c * 16, 16)
                rowbase = (lax.iota(jnp.int32, 16) + c * 16) * _NROWS
                idx_v[sl] = rowbase + lax.shift_right_logical(lab_v[sl], 4)
            pltpu.async_copy(t1_hbm.at[idx_v], r1_v, sem).wait()
            pltpu.async_copy(t2_hbm.at[idx_v], r2_v, sem).wait()
            pltpu.sync_copy(r1_v, o1_hbm)
            pltpu.sync_copy(r2_v, o2_hbm)

    return k(lab1d, tab1, tab2)


def _select_kernel(rr_ref, uf_ref, l1c_ref, l2c_ref, dc_ref,
                   l1r_ref, l2r_ref, dr_ref,
                   tg1_ref, tg2_ref, tgT1_ref, tgT2_ref,
                   lm_c_ref, lm_r_ref, o1_ref, o2_ref):
    i16r = jax.lax.broadcasted_iota(jnp.int32, (_B, _LL), 1)
    lm_c = lm_c_ref[...]
    t1c = jnp.sum(jnp.where(i16r == lm_c, tg1_ref[...], 0.0),
                  axis=1, keepdims=True)
    t2c = jnp.sum(jnp.where(i16r == lm_c, tg2_ref[...], 0.0),
                  axis=1, keepdims=True)
    i16c = jax.lax.broadcasted_iota(jnp.int32, (_LL, _B), 0)
    lm_r = lm_r_ref[0:1, :]
    t1r = jnp.sum(jnp.where(i16c == lm_r, tgT1_ref[...], 0.0),
                  axis=0, keepdims=True)
    t2r = jnp.sum(jnp.where(i16c == lm_r, tgT2_ref[...], 0.0),
                  axis=0, keepdims=True)
    l1c = l1c_ref[...] - t1c
    l2c = l2c_ref[...] - t2c
    l1r = l1r_ref[0:1, :] - t1r
    l2r = l2r_ref[0:1, :] - t2r
    dc = dc_ref[...]
    dr = dr_ref[0:1, :]

    dcf = dc.astype(jnp.float32)
    drf = dr.astype(jnp.float32)
    D = jnp.sum(dcf)
    ridc = jax.lax.broadcasted_iota(jnp.int32, (_B, 1), 0)
    dropped = jnp.sum(jnp.where(ridc == 0, dcf, 0.0))
    L = D - dropped

    rid = jax.lax.broadcasted_iota(jnp.int32, (_B, _B), 0)
    cid = jax.lax.broadcasted_iota(jnp.int32, (_B, _B), 1)
    dr2 = jnp.broadcast_to(drf, (_B, _B))
    dc2 = jnp.broadcast_to(dcf, (_B, _B))
    pref_c = jnp.sum(jnp.where(cid < rid, dr2, 0.0), axis=1, keepdims=True)
    pref_r = jnp.sum(jnp.where(rid < cid, dc2, 0.0), axis=0, keepdims=True)
    sel_c = (dc != 0) & (pref_c < L)
    sel_r = (dr != 0) & (pref_r < L)
    sel_r2 = jnp.broadcast_to(sel_r, (_B, _B))

    cmp2 = (l2r < l2c) | ((l2r == l2c) & (cid < rid))
    rank2 = jnp.sum(jnp.where(cmp2 & sel_r2, 1.0, 0.0), axis=1, keepdims=True)
    cmp1 = (l1r < l1c) | ((l1r == l1c) & (cid < rid))
    rank1 = jnp.sum(jnp.where(cmp1 & sel_r2, 1.0, 0.0), axis=1, keepdims=True)

    rr = rr_ref[0, 0]
    nr = jnp.floor(rr * L)
    k = jnp.where(nr == 0.0, L, nr)
    keep2 = sel_c & (rank2 < k)
    keep1 = sel_c & (rank1 < k)
    loss1_upd = jnp.sum(jnp.where(keep2, l1c, 0.0)) / k
    loss2_upd = jnp.sum(jnp.where(keep1, l2c, 0.0)) / k

    uf = uf_ref[0, 0]
    us = jnp.where((dc != 0) | (uf != 0), 1.0, 0.0)
    fb1 = jnp.sum(us * l1c) / _B
    fb2 = jnp.sum(us * l2c) / _B

    o1_ref[0, 0] = jnp.where(L > 0, loss1_upd, fb1)
    o2_ref[0, 0] = jnp.where(L > 0, loss2_upd, fb2)


def _select_call(rr, uf, lse1, lse2, dis, lse1r, lse2r, disr,
                 tg1, tg2, tgT1, tgT2, lm_c, lm_r):
    o1, o2 = pl.pallas_call(
        _select_kernel,
        in_specs=[
            pl.BlockSpec(memory_space=pltpu.SMEM),
            pl.BlockSpec(memory_space=pltpu.SMEM),
            pl.BlockSpec((_B, 1), lambda: (0, 0)),
            pl.BlockSpec((_B, 1), lambda: (0, 0)),
            pl.BlockSpec((_B, 1), lambda: (0, 0)),
            pl.BlockSpec((8, _B), lambda: (0, 0)),
            pl.BlockSpec((8, _B), lambda: (0, 0)),
            pl.BlockSpec((8, _B), lambda: (0, 0)),
            pl.BlockSpec((_B, _LL), lambda: (0, 0)),
            pl.BlockSpec((_B, _LL), lambda: (0, 0)),
            pl.BlockSpec((_LL, _B), lambda: (0, 0)),
            pl.BlockSpec((_LL, _B), lambda: (0, 0)),
            pl.BlockSpec((_B, 1), lambda: (0, 0)),
            pl.BlockSpec((8, _B), lambda: (0, 0)),
        ],
        out_specs=[
            pl.BlockSpec(memory_space=pltpu.SMEM),
            pl.BlockSpec(memory_space=pltpu.SMEM),
        ],
        out_shape=[
            jax.ShapeDtypeStruct((1, 1), jnp.float32),
            jax.ShapeDtypeStruct((1, 1), jnp.float32),
        ],
    )(rr, uf, lse1, lse2, dis, lse1r, lse2r, disr,
      tg1, tg2, tgT1, tgT2, lm_c, lm_r)
    return o1, o2


def kernel(logits, logits2, labels, epoch, ind, step):
    lab = labels.astype(jnp.int32)
    lse1, lse2, dis = _stats_call(logits, logits2)
    tg1, tg2 = _gather_call(lab, logits.reshape(_B * _NROWS, _LL),
                            logits2.reshape(_B * _NROWS, _LL))
    rr = (1.0 - jnp.asarray(_SCHED)[epoch]).astype(jnp.float32).reshape(1, 1)
    uf = (jnp.asarray(step) < 5000).astype(jnp.int32).reshape(1, 1)
    lse1r = jnp.broadcast_to(lse1.reshape(1, _B), (8, _B))
    lse2r = jnp.broadcast_to(lse2.reshape(1, _B), (8, _B))
    disr = jnp.broadcast_to(dis.reshape(1, _B), (8, _B))
    tgT1 = jnp.swapaxes(tg1, 0, 1)
    tgT2 = jnp.swapaxes(tg2, 0, 1)
    lm = lab & (_LL - 1)
    lm_c = lm.reshape(_B, 1)
    lm_r = jnp.broadcast_to(lm.reshape(1, _B), (8, _B))
    o1, o2 = _select_call(rr, uf, lse1, lse2, dis, lse1r, lse2r, disr,
                          tg1, tg2, tgT1, tgT2, lm_c, lm_r)
    return o1.reshape(()), o2.reshape(())


# speedup vs baseline: 3.5192x; 1.8346x over previous
import functools

import jax
import jax.numpy as jnp
import numpy as np
from jax import lax
from jax.experimental import pallas as pl
from jax.experimental.pallas import tpu as pltpu
from jax.experimental.pallas import tpu_sc as plsc

_FORGET_RATE = 0.2
_NUM_GRADUAL = 5
_N_EPOCH = 10
_SCHED = np.ones(_N_EPOCH, np.float32) * _FORGET_RATE
_SCHED[:_NUM_GRADUAL] = np.linspace(0.0, _FORGET_RATE, _NUM_GRADUAL)

_B = 128
_V = 100000
_BV = 8192
_NBLK = -(-_V // _BV)
_LL = 128
_ATAIL = (_V // _LL) * _LL
_TS = _V - _LL


def _stats_kernel(x1_ref, x2_ref, lse1_ref, lse2_ref, dis_ref,
                  m1, s1, a1, m2, s2, a2):
    j = pl.program_id(0)

    @pl.when(j == 0)
    def _init():
        for m, s, a in ((m1, s1, a1), (m2, s2, a2)):
            m[...] = jnp.full((_B, 1), -jnp.inf, jnp.float32)
            s[...] = jnp.zeros((_B, 1), jnp.float32)
            a[...] = jnp.zeros((_B, 1), jnp.int32)

    col = j * _BV + jax.lax.broadcasted_iota(jnp.int32, (_B, _BV), 1)

    def upd(x_raw, m, s, a):
        x = jnp.where(col < _V, x_raw, -jnp.inf)
        bmax = jnp.max(x, axis=1, keepdims=True)
        bidx = jnp.min(jnp.where(x == bmax, col, _V), axis=1, keepdims=True)
        m_old = m[...]
        m_new = jnp.maximum(m_old, bmax)
        s[...] = s[...] * jnp.exp(m_old - m_new) + jnp.sum(
            jnp.exp(x - m_new), axis=1, keepdims=True)
        a[...] = jnp.where(bmax > m_old, bidx, a[...])
        m[...] = m_new

    upd(x1_ref[...], m1, s1, a1)
    upd(x2_ref[...], m2, s2, a2)

    @pl.when(j == _NBLK - 1)
    def _fin():
        lse1_ref[...] = m1[...] + jnp.log(s1[...])
        lse2_ref[...] = m2[...] + jnp.log(s2[...])
        dis_ref[...] = (a1[...] != a2[...]).astype(jnp.int32)


def _stats_call(logits, logits2):
    return pl.pallas_call(
        _stats_kernel,
        grid=(_NBLK,),
        in_specs=[
            pl.BlockSpec((_B, _BV), lambda j: (0, j)),
            pl.BlockSpec((_B, _BV), lambda j: (0, j)),
        ],
        out_specs=[
            pl.BlockSpec((_B, 1), lambda j: (0, 0)),
            pl.BlockSpec((_B, 1), lambda j: (0, 0)),
            pl.BlockSpec((_B, 1), lambda j: (0, 0)),
        ],
        out_shape=[
            jax.ShapeDtypeStruct((_B, 1), jnp.float32),
            jax.ShapeDtypeStruct((_B, 1), jnp.float32),
            jax.ShapeDtypeStruct((_B, 1), jnp.int32),
        ],
        scratch_shapes=[
            pltpu.VMEM((_B, 1), jnp.float32),
            pltpu.VMEM((_B, 1), jnp.float32),
            pltpu.VMEM((_B, 1), jnp.int32),
            pltpu.VMEM((_B, 1), jnp.float32),
            pltpu.VMEM((_B, 1), jnp.float32),
            pltpu.VMEM((_B, 1), jnp.int32),
        ],
        compiler_params=pltpu.CompilerParams(
            dimension_semantics=("arbitrary",)),
    )(logits, logits2)


def _gather_call(lab1d, tab1, tab2, tt1, tt2):
    mesh = plsc.VectorSubcoreMesh(core_axis_name="c", subcore_axis_name="s")

    @functools.partial(
        pl.kernel, mesh=mesh,
        out_type=[jax.ShapeDtypeStruct((_B, _LL), jnp.float32),
                  jax.ShapeDtypeStruct((_B, _LL), jnp.float32)],
        scratch_types=[
            pltpu.VMEM((_B,), jnp.int32),
            pltpu.VMEM((8, 8, _LL), jnp.float32),
            pltpu.VMEM((8, 8, _LL), jnp.float32),
            pltpu.VMEM((8, _LL), jnp.float32),
            pltpu.VMEM((8, _LL), jnp.float32),
            pltpu.SemaphoreType.DMA,
        ],
    )
    def k(lab_hbm, t1_hbm, t2_hbm, tt1_hbm, tt2_hbm, o1_hbm, o2_hbm,
          lab_v, st1_v, st2_v, w1_v, w2_v, sem):
        cid = lax.axis_index("c")
        sid = lax.axis_index("s")
        wid = sid * 2 + cid

        @pl.when(wid < 16)
        def _():
            pltpu.sync_copy(lab_hbm, lab_v)
            rbase = pl.multiple_of(wid * 8, 8)
            lvec = lab_v[pl.ds(pl.multiple_of((wid // 2) * 16, 16), 16)]
            avec = lvec & ~(_LL - 1)

            def issue(e0):
                for e in range(8):
                    acol = avec[e0 + e]

                    @pl.when(acol != _ATAIL)
                    def _reg(acol=acol, e=e):
                        a = pl.multiple_of(acol, _LL)
                        cp1 = pltpu.make_async_copy(
                            t1_hbm.at[pl.ds(rbase, 8), pl.ds(a, _LL)],
                            st1_v.at[e], sem)
                        cp2 = pltpu.make_async_copy(
                            t2_hbm.at[pl.ds(rbase, 8), pl.ds(a, _LL)],
                            st2_v.at[e], sem)
                        cp1.start()
                        cp2.start()
                        cp1.wait()
                        cp2.wait()

                    @pl.when(acol == _ATAIL)
                    def _tail(e=e):
                        cp1 = pltpu.make_async_copy(
                            tt1_hbm.at[pl.ds(rbase, 8)],
                            st1_v.at[e], sem)
                        cp2 = pltpu.make_async_copy(
                            tt2_hbm.at[pl.ds(rbase, 8)],
                            st2_v.at[e], sem)
                        cp1.start()
                        cp2.start()
                        cp1.wait()
                        cp2.wait()

            @pl.when(wid % 2 == 0)
            def _even():
                issue(0)

            @pl.when(wid % 2 == 1)
            def _odd():
                issue(8)

            for e in range(8):
                for v in range(_LL // 16):
                    sl = pl.ds(v * 16, 16)
                    w1_v[e, sl] = st1_v[e, e, sl]
                    w2_v[e, sl] = st2_v[e, e, sl]
            pltpu.sync_copy(w1_v, o1_hbm.at[pl.ds(rbase, 8)])
            pltpu.sync_copy(w2_v, o2_hbm.at[pl.ds(rbase, 8)])

    return k(lab1d, tab1, tab2, tt1, tt2)


def _select_kernel(rr_ref, uf_ref, l1c_ref, l2c_ref, dc_ref,
                   l1r_ref, l2r_ref, dr_ref,
                   tg1_ref, tg2_ref, tgT1_ref, tgT2_ref,
                   lm_c_ref, lm_r_ref, o1_ref, o2_ref):
    i16r = jax.lax.broadcasted_iota(jnp.int32, (_B, _LL), 1)
    lm_c = lm_c_ref[...]
    t1c = jnp.sum(jnp.where(i16r == lm_c, tg1_ref[...], 0.0),
                  axis=1, keepdims=True)
    t2c = jnp.sum(jnp.where(i16r == lm_c, tg2_ref[...], 0.0),
                  axis=1, keepdims=True)
    i16c = jax.lax.broadcasted_iota(jnp.int32, (_LL, _B), 0)
    lm_r = lm_r_ref[0:1, :]
    t1r = jnp.sum(jnp.where(i16c == lm_r, tgT1_ref[...], 0.0),
                  axis=0, keepdims=True)
    t2r = jnp.sum(jnp.where(i16c == lm_r, tgT2_ref[...], 0.0),
                  axis=0, keepdims=True)
    l1c = l1c_ref[...] - t1c
    l2c = l2c_ref[...] - t2c
    l1r = l1r_ref[0:1, :] - t1r
    l2r = l2r_ref[0:1, :] - t2r
    dc = dc_ref[...]
    dr = dr_ref[0:1, :]

    dcf = dc.astype(jnp.float32)
    drf = dr.astype(jnp.float32)
    D = jnp.sum(dcf)
    ridc = jax.lax.broadcasted_iota(jnp.int32, (_B, 1), 0)
    dropped = jnp.sum(jnp.where(ridc == 0, dcf, 0.0))
    L = D - dropped

    rid = jax.lax.broadcasted_iota(jnp.int32, (_B, _B), 0)
    cid = jax.lax.broadcasted_iota(jnp.int32, (_B, _B), 1)
    dr2 = jnp.broadcast_to(drf, (_B, _B))
    dc2 = jnp.broadcast_to(dcf, (_B, _B))
    pref_c = jnp.sum(jnp.where(cid < rid, dr2, 0.0), axis=1, keepdims=True)
    pref_r = jnp.sum(jnp.where(rid < cid, dc2, 0.0), axis=0, keepdims=True)
    sel_c = (dc != 0) & (pref_c < L)
    sel_r = (dr != 0) & (pref_r < L)
    sel_r2 = jnp.broadcast_to(sel_r, (_B, _B))

    cmp2 = (l2r < l2c) | ((l2r == l2c) & (cid < rid))
    rank2 = jnp.sum(jnp.where(cmp2 & sel_r2, 1.0, 0.0), axis=1, keepdims=True)
    cmp1 = (l1r < l1c) | ((l1r == l1c) & (cid < rid))
    rank1 = jnp.sum(jnp.where(cmp1 & sel_r2, 1.0, 0.0), axis=1, keepdims=True)

    rr = rr_ref[0, 0]
    nr = jnp.floor(rr * L)
    k = jnp.where(nr == 0.0, L, nr)
    keep2 = sel_c & (rank2 < k)
    keep1 = sel_c & (rank1 < k)
    loss1_upd = jnp.sum(jnp.where(keep2, l1c, 0.0)) / k
    loss2_upd = jnp.sum(jnp.where(keep1, l2c, 0.0)) / k

    uf = uf_ref[0, 0]
    us = jnp.where((dc != 0) | (uf != 0), 1.0, 0.0)
    fb1 = jnp.sum(us * l1c) / _B
    fb2 = jnp.sum(us * l2c) / _B

    o1_ref[0, 0] = jnp.where(L > 0, loss1_upd, fb1)
    o2_ref[0, 0] = jnp.where(L > 0, loss2_upd, fb2)


def _select_call(rr, uf, lse1, lse2, dis, lse1r, lse2r, disr,
                 tg1, tg2, tgT1, tgT2, lm_c, lm_r):
    o1, o2 = pl.pallas_call(
        _select_kernel,
        in_specs=[
            pl.BlockSpec(memory_space=pltpu.SMEM),
            pl.BlockSpec(memory_space=pltpu.SMEM),
            pl.BlockSpec((_B, 1), lambda: (0, 0)),
            pl.BlockSpec((_B, 1), lambda: (0, 0)),
            pl.BlockSpec((_B, 1), lambda: (0, 0)),
            pl.BlockSpec((8, _B), lambda: (0, 0)),
            pl.BlockSpec((8, _B), lambda: (0, 0)),
            pl.BlockSpec((8, _B), lambda: (0, 0)),
            pl.BlockSpec((_B, _LL), lambda: (0, 0)),
            pl.BlockSpec((_B, _LL), lambda: (0, 0)),
            pl.BlockSpec((_LL, _B), lambda: (0, 0)),
            pl.BlockSpec((_LL, _B), lambda: (0, 0)),
            pl.BlockSpec((_B, 1), lambda: (0, 0)),
            pl.BlockSpec((8, _B), lambda: (0, 0)),
        ],
        out_specs=[
            pl.BlockSpec(memory_space=pltpu.SMEM),
            pl.BlockSpec(memory_space=pltpu.SMEM),
        ],
        out_shape=[
            jax.ShapeDtypeStruct((1, 1), jnp.float32),
            jax.ShapeDtypeStruct((1, 1), jnp.float32),
        ],
    )(rr, uf, lse1, lse2, dis, lse1r, lse2r, disr,
      tg1, tg2, tgT1, tgT2, lm_c, lm_r)
    return o1, o2


def kernel(logits, logits2, labels, epoch, ind, step):
    lab = labels.astype(jnp.int32)
    lse1, lse2, dis = _stats_call(logits, logits2)
    tg1, tg2 = _gather_call(lab, logits, logits2,
                            logits[:, _TS:], logits2[:, _TS:])
    rr = (1.0 - jnp.asarray(_SCHED)[epoch]).astype(jnp.float32).reshape(1, 1)
    uf = (jnp.asarray(step) < 5000).astype(jnp.int32).reshape(1, 1)
    lse1r = jnp.broadcast_to(lse1.reshape(1, _B), (8, _B))
    lse2r = jnp.broadcast_to(lse2.reshape(1, _B), (8, _B))
    disr = jnp.broadcast_to(dis.reshape(1, _B), (8, _B))
    tgT1 = jnp.swapaxes(tg1, 0, 1)
    tgT2 = jnp.swapaxes(tg2, 0, 1)
    lm = jnp.where((lab & ~(_LL - 1)) == _ATAIL, lab - _TS, lab & (_LL - 1))
    lm_c = lm.reshape(_B, 1)
    lm_r = jnp.broadcast_to(lm.reshape(1, _B), (8, _B))
    o1, o2 = _select_call(rr, uf, lse1, lse2, dis, lse1r, lse2r, disr,
                          tg1, tg2, tgT1, tgT2, lm_c, lm_r)
    return o1.reshape(()), o2.reshape(())
